# baseline (speedup 1.0000x reference)
import jax
import jax.numpy as jnp
from jax.experimental import pallas as pl
from jax.experimental.pallas import tpu as pltpu


def _make_body(b):
    def _body(col_ref, row_ref, o_ref, scratch, sems):
        w, d = col_ref.shape
        h = row_ref.shape[0]
        scratch[:, :, :d] = jnp.broadcast_to(col_ref[...][None, :, :], (h, w, d))
        col_copies = [
            pltpu.make_async_copy(
                scratch.at[:, :, :d], o_ref.at[i, :, :, :d], sems.at[i]
            )
            for i in range(b)
        ]
        for c in col_copies:
            c.start()
        scratch[:, :, d:] = jnp.broadcast_to(row_ref[...][:, None, :], (h, w, d))
        row_copies = [
            pltpu.make_async_copy(
                scratch.at[:, :, d:], o_ref.at[i, :, :, d:], sems.at[b + i]
            )
            for i in range(b)
        ]
        for c in row_copies:
            c.start()
        for c in col_copies:
            c.wait()
        for c in row_copies:
            c.wait()

    return _body


def kernel(x, mask, row_embed, col_embed):
    b = x.shape[0]
    h, w = x.shape[-2], x.shape[-1]
    d = col_embed.shape[-1]
    out_nat = pl.pallas_call(
        _make_body(b),
        grid=(1,),
        in_specs=[
            pl.BlockSpec((w, d), lambda i: (0, 0)),
            pl.BlockSpec((h, d), lambda i: (0, 0)),
        ],
        out_specs=pl.BlockSpec(memory_space=pl.ANY),
        out_shape=jax.ShapeDtypeStruct((b, h, w, 2 * d), jnp.float32),
        scratch_shapes=[
            pltpu.VMEM((h, w, 2 * d), jnp.float32),
            pltpu.SemaphoreType.DMA((2 * b,)),
        ],
    )(col_embed, row_embed)
    return jnp.transpose(out_nat, (0, 3, 1, 2))
